# baseline (speedup 1.0000x reference)
import functools

import jax
import jax.numpy as jnp
from jax import lax
from jax.experimental import pallas as pl
from jax.experimental.pallas import tpu as pltpu

S = 2048
DM = 1024
NH = 16
HD = 64
EPS = 1e-6
RB = 256
CLASSES = ((0, 256, 256), (256, 256, 512), (512, 512, 1024), (1024, 1024, 2048))
RBLK = {256: 64, 512: 64, 1024: 32, 2048: 32}


def _proj_kernel(x_ref, wq_ref, wk_ref, ww_ref, q_ref, k_ref, w_ref):
    x = x_ref[...]
    q_ref[...] = jnp.dot(x, wq_ref[...], preferred_element_type=jnp.float32)
    k_ref[...] = jnp.dot(x, wk_ref[...], preferred_element_type=jnp.float32)
    w_ref[...] = jnp.dot(x, ww_ref[...], preferred_element_type=jnp.float32)


def _score_kernel(q_ref, w_ref, kr_ref, o_ref):
    i = pl.program_id(0)
    kr = kr_ref[...]
    acc = jnp.zeros((RB, S), jnp.float32)
    for n in range(NH):
        qn = q_ref[:, n * HD:(n + 1) * HD]
        sn = lax.dot_general(qn, kr, (((1,), (1,)), ((), ())),
                             preferred_element_type=jnp.float32)
        acc = acc + jnp.maximum(sn, 0.0) * w_ref[:, n:n + 1]
    row = i * RB + lax.broadcasted_iota(jnp.int32, (RB, S), 0)
    col = lax.broadcasted_iota(jnp.int32, (RB, S), 1)
    maskval = -1e30 - col.astype(jnp.float32) * 1e24
    o_ref[...] = jnp.where(col > row, maskval, acc)


def _xor_perm(a, d, width):
    parts = []
    for blk in range(0, width, 2 * d):
        parts.append(a[:, blk + d:blk + 2 * d])
        parts.append(a[:, blk:blk + d])
    return jnp.concatenate(parts, axis=1)


def _ce_stage(key, idx, pos, j, lodesc, width):
    d = 1 << j
    if d >= 128:
        pk = _xor_perm(key, d, width)
        pi = _xor_perm(idx, d, width)
    else:
        is_lo = (pos & d) == 0
        km = pltpu.roll(key, width - d, 1)
        kp = pltpu.roll(key, d, 1)
        im = pltpu.roll(idx, width - d, 1)
        ip = pltpu.roll(idx, d, 1)
        pk = jnp.where(is_lo, km, kp)
        pi = jnp.where(is_lo, im, ip)
    keep = (key > pk) ^ lodesc
    return jnp.where(keep, key, pk), jnp.where(keep, idx, pi)


def _make_sort(w, nrows):
    logw = w.bit_length() - 1

    def body(sc_ref, o_ref):
        pos = lax.broadcasted_iota(jnp.int32, (nrows, w), 1)
        key = sc_ref[...]
        idx = pos
        for k in range(1, logw + 1):
            desc = ((pos & (1 << k)) == 0) if k < logw else None
            for j in reversed(range(k)):
                is_lo = (pos & (1 << j)) == 0
                lodesc = ~is_lo if desc is None else is_lo ^ desc
                key, idx = _ce_stage(key, idx, pos, j, lodesc, w)
        o_ref[...] = idx

    return body


def _rms_norm(x, w, eps=EPS):
    var = jnp.mean(x * x, axis=-1, keepdims=True)
    return x * jax.lax.rsqrt(var + eps) * w


def _rotate_half(x):
    h = x.shape[-1] // 2
    return jnp.concatenate([-x[..., h:], x[..., :h]], axis=-1)


@jax.jit
def kernel(x, cos, sin, Wq, Wk, Ww, q_norm_w, k_norm_w, start_pos, end_pos,
           use_cache):
    del start_pos, end_pos, use_cache
    b = x.shape[0]
    x2 = x[0]

    q2draw, kraw, wraw = pl.pallas_call(
        _proj_kernel,
        grid=(S // RB,),
        in_specs=[
            pl.BlockSpec((RB, DM), lambda i: (i, 0)),
            pl.BlockSpec((DM, DM), lambda i: (0, 0)),
            pl.BlockSpec((DM, HD), lambda i: (0, 0)),
            pl.BlockSpec((DM, NH), lambda i: (0, 0)),
        ],
        out_specs=(pl.BlockSpec((RB, DM), lambda i: (i, 0)),
                   pl.BlockSpec((RB, HD), lambda i: (i, 0)),
                   pl.BlockSpec((RB, NH), lambda i: (i, 0))),
        out_shape=(jax.ShapeDtypeStruct((S, DM), jnp.float32),
                   jax.ShapeDtypeStruct((S, HD), jnp.float32),
                   jax.ShapeDtypeStruct((S, NH), jnp.float32)),
    )(x2, Wq, Wk, Ww)

    q = _rms_norm(q2draw.reshape(b, S, NH, HD), q_norm_w)
    k = _rms_norm(kraw.reshape(b, S, 1, HD), k_norm_w)
    c4 = cos[:, :, None, :]
    s4 = sin[:, :, None, :]
    q = q * c4 + _rotate_half(q) * s4
    k = k * c4 + _rotate_half(k) * s4
    k = k[:, :, 0, :]
    weights = wraw.reshape(b, S, NH) * (NH ** -0.5)
    weights = weights * (HD ** -0.5)

    q2d = q.reshape(S, NH * HD)
    w2d = weights[0]
    kr = k[0]

    scores = pl.pallas_call(
        _score_kernel,
        grid=(S // RB,),
        in_specs=[
            pl.BlockSpec((RB, NH * HD), lambda i: (i, 0)),
            pl.BlockSpec((RB, NH), lambda i: (i, 0)),
            pl.BlockSpec((S, HD), lambda i: (0, 0)),
        ],
        out_specs=pl.BlockSpec((RB, S), lambda i: (i, 0)),
        out_shape=jax.ShapeDtypeStruct((S, S), jnp.float32),
    )(q2d, w2d, kr)

    pieces = []
    for (r0, rc, w) in CLASSES:
        nb = RBLK[w]
        blk0 = r0 // nb
        sorted_idx = pl.pallas_call(
            _make_sort(w, nb),
            grid=(rc // nb,),
            in_specs=[pl.BlockSpec((nb, w), lambda i, b0=blk0: (b0 + i, 0))],
            out_specs=pl.BlockSpec((nb, w), lambda i: (i, 0)),
            out_shape=jax.ShapeDtypeStruct((rc, w), jnp.int32),
        )(scores)
        if w < S:
            tail = jnp.broadcast_to(
                jnp.arange(w, S, dtype=jnp.int32)[None, :], (rc, S - w))
            sorted_idx = jnp.concatenate([sorted_idx, tail], axis=1)
        pieces.append(sorted_idx)

    idx = jnp.concatenate(pieces, axis=0)
    return idx.reshape(1, S, S)


# speedup vs baseline: 1.0776x; 1.0776x over previous
import functools

import jax
import jax.numpy as jnp
from jax import lax
from jax.experimental import pallas as pl
from jax.experimental.pallas import tpu as pltpu

S = 2048
DM = 1024
NH = 16
HD = 64
EPS = 1e-6
RB = 256
CLASSES = ((0, 256, 256), (256, 256, 512), (512, 512, 1024), (1024, 1024, 2048))
RBLK = {256: 64, 512: 64, 1024: 32, 2048: 32}


def _proj_kernel(x_ref, wq_ref, wk_ref, ww_ref, q_ref, k_ref, w_ref):
    x = x_ref[...]
    q_ref[...] = jnp.dot(x, wq_ref[...], preferred_element_type=jnp.float32)
    k_ref[...] = jnp.dot(x, wk_ref[...], preferred_element_type=jnp.float32)
    w_ref[...] = jnp.dot(x, ww_ref[...], preferred_element_type=jnp.float32)


def _score_kernel(q_ref, w_ref, kr_ref, o_ref):
    i = pl.program_id(0)
    kr = kr_ref[...]
    acc = jnp.zeros((RB, S), jnp.float32)
    for n in range(NH):
        qn = q_ref[:, n * HD:(n + 1) * HD]
        sn = lax.dot_general(qn, kr, (((1,), (1,)), ((), ())),
                             preferred_element_type=jnp.float32)
        acc = acc + jnp.maximum(sn, 0.0) * w_ref[:, n:n + 1]
    row = i * RB + lax.broadcasted_iota(jnp.int32, (RB, S), 0)
    col = lax.broadcasted_iota(jnp.int32, (RB, S), 1)
    maskval = -1e30 - col.astype(jnp.float32) * 1e24
    o_ref[...] = jnp.where(col > row, maskval, acc)


def _xor_perm(a, d, width):
    parts = []
    for blk in range(0, width, 2 * d):
        parts.append(a[:, blk + d:blk + 2 * d])
        parts.append(a[:, blk:blk + d])
    return jnp.concatenate(parts, axis=1)


def _ce_stage(key, idx, pos, j, lodesc, width):
    d = 1 << j
    if d >= 128:
        pk = _xor_perm(key, d, width)
        pi = _xor_perm(idx, d, width)
    else:
        is_lo = (pos & d) == 0
        km = pltpu.roll(key, width - d, 1)
        kp = pltpu.roll(key, d, 1)
        im = pltpu.roll(idx, width - d, 1)
        ip = pltpu.roll(idx, d, 1)
        pk = jnp.where(is_lo, km, kp)
        pi = jnp.where(is_lo, im, ip)
    keep = (key > pk) ^ lodesc
    return jnp.where(keep, key, pk), jnp.where(keep, idx, pi)


def _make_sort(w, nrows):
    logw = w.bit_length() - 1

    def body(sc_ref, o_ref):
        pos = lax.broadcasted_iota(jnp.int32, (nrows, w), 1)
        key = sc_ref[...]
        idx = pos
        for k in range(1, logw + 1):
            desc = ((pos & (1 << k)) == 0) if k < logw else None
            for j in reversed(range(k)):
                is_lo = (pos & (1 << j)) == 0
                lodesc = ~is_lo if desc is None else is_lo ^ desc
                key, idx = _ce_stage(key, idx, pos, j, lodesc, w)
        o_ref[...] = idx

    return body


def _rms_norm(x, w, eps=EPS):
    var = jnp.mean(x * x, axis=-1, keepdims=True)
    return x * jax.lax.rsqrt(var + eps) * w


def _rotate_half(x):
    h = x.shape[-1] // 2
    return jnp.concatenate([-x[..., h:], x[..., :h]], axis=-1)


@jax.jit
def kernel(x, cos, sin, Wq, Wk, Ww, q_norm_w, k_norm_w, start_pos, end_pos,
           use_cache):
    del start_pos, end_pos, use_cache
    b = x.shape[0]
    x2 = x[0]

    q2draw, kraw, wraw = pl.pallas_call(
        _proj_kernel,
        grid=(S // RB,),
        in_specs=[
            pl.BlockSpec((RB, DM), lambda i: (i, 0)),
            pl.BlockSpec((DM, DM), lambda i: (0, 0)),
            pl.BlockSpec((DM, HD), lambda i: (0, 0)),
            pl.BlockSpec((DM, NH), lambda i: (0, 0)),
        ],
        out_specs=(pl.BlockSpec((RB, DM), lambda i: (i, 0)),
                   pl.BlockSpec((RB, HD), lambda i: (i, 0)),
                   pl.BlockSpec((RB, NH), lambda i: (i, 0))),
        out_shape=(jax.ShapeDtypeStruct((S, DM), jnp.float32),
                   jax.ShapeDtypeStruct((S, HD), jnp.float32),
                   jax.ShapeDtypeStruct((S, NH), jnp.float32)),
    )(x2, Wq, Wk, Ww)

    q = _rms_norm(q2draw.reshape(b, S, NH, HD), q_norm_w)
    k = _rms_norm(kraw.reshape(b, S, 1, HD), k_norm_w)
    q = jnp.transpose(q, (0, 2, 1, 3))
    k = jnp.transpose(k, (0, 2, 1, 3))
    c4 = cos[:, None, :, :]
    s4 = sin[:, None, :, :]
    q = q * c4 + _rotate_half(q) * s4
    k = k * c4 + _rotate_half(k) * s4
    q = jnp.transpose(q, (0, 2, 1, 3))
    k = jnp.transpose(k, (0, 2, 1, 3))[:, :, 0, :]
    weights = wraw.reshape(b, S, NH) * (NH ** -0.5)
    weights = weights * (HD ** -0.5)

    q2d = q.reshape(S, NH * HD)
    w2d = weights[0]
    kr = k[0]

    scores = pl.pallas_call(
        _score_kernel,
        grid=(S // RB,),
        in_specs=[
            pl.BlockSpec((RB, NH * HD), lambda i: (i, 0)),
            pl.BlockSpec((RB, NH), lambda i: (i, 0)),
            pl.BlockSpec((S, HD), lambda i: (0, 0)),
        ],
        out_specs=pl.BlockSpec((RB, S), lambda i: (i, 0)),
        out_shape=jax.ShapeDtypeStruct((S, S), jnp.float32),
    )(q2d, w2d, kr)

    pieces = []
    for (r0, rc, w) in CLASSES:
        nb = RBLK[w]
        blk0 = r0 // nb
        sorted_idx = pl.pallas_call(
            _make_sort(w, nb),
            grid=(rc // nb,),
            in_specs=[pl.BlockSpec((nb, w), lambda i, b0=blk0: (b0 + i, 0))],
            out_specs=pl.BlockSpec((nb, w), lambda i: (i, 0)),
            out_shape=jax.ShapeDtypeStruct((rc, w), jnp.int32),
        )(scores)
        if w < S:
            tail = jnp.broadcast_to(
                jnp.arange(w, S, dtype=jnp.int32)[None, :], (rc, S - w))
            sorted_idx = jnp.concatenate([sorted_idx, tail], axis=1)
        pieces.append(sorted_idx)

    idx = jnp.concatenate(pieces, axis=0)
    return idx.reshape(1, S, S)
